# baseline (speedup 1.0000x reference)
import jax
import jax.numpy as jnp
from jax.experimental import pallas as pl
from jax.experimental.pallas import tpu as pltpu

NUM_EXPERTS = 64
TOP_K = 8
HIDDEN = 4096
BATCH = 2
SEQ = 4096
TOKENS = BATCH * SEQ
TILE_T = 1024
SUB_T = 128


def _router_body(x_ref, w_ref, vals_ref, idx_ref, aux_ref,
                 cnt_ref, psum_ref, lg_ref):
    i = pl.program_id(0)

    @pl.when(i == 0)
    def _init():
        cnt_ref[...] = jnp.zeros_like(cnt_ref)
        psum_ref[...] = jnp.zeros_like(psum_ref)

    lg_ref[...] = jax.lax.dot_general(
        w_ref[...], x_ref[...], (((1,), (1,)), ((), ())),
        preferred_element_type=jnp.float32)

    psum_acc = jnp.zeros((NUM_EXPERTS, SUB_T), jnp.float32)
    cnt_acc = jnp.zeros((NUM_EXPERTS, SUB_T), jnp.float32)

    for c in range(TILE_T // SUB_T):
        sl = pl.ds(c * SUB_T, SUB_T)
        logits = lg_ref[:, sl]
        m = jnp.max(logits, axis=0, keepdims=True)
        e = jnp.exp(logits - m)
        s = jnp.sum(e, axis=0, keepdims=True)
        probs = e / s
        psum_acc = psum_acc + probs

        row_f = jax.lax.broadcasted_iota(jnp.int32, probs.shape, 0).astype(jnp.float32)
        work = probs
        vals_rows = []
        idx_rows = []
        for _ in range(TOP_K):
            mk = jnp.max(work, axis=0, keepdims=True)
            ik = jnp.min(jnp.where(work == mk, row_f, jnp.float32(NUM_EXPERTS)),
                         axis=0, keepdims=True)
            vals_rows.append(mk)
            idx_rows.append(ik)
            work = jnp.where(row_f == ik, -1.0, work)

        cnt_acc = cnt_acc + jnp.where(work < 0.0, 1.0, 0.0)

        valsT = jnp.concatenate(vals_rows, axis=0)
        idxT = jnp.concatenate(idx_rows, axis=0)
        valsT = valsT / jnp.sum(valsT, axis=0, keepdims=True)
        vals_ref[sl, :] = valsT.T
        idx_ref[sl, :] = idxT.T.astype(jnp.int32)

    psum_ref[...] += jnp.sum(psum_acc, axis=1, keepdims=True)
    cnt_ref[...] += jnp.sum(cnt_acc, axis=1, keepdims=True)

    @pl.when(i == pl.num_programs(0) - 1)
    def _fin():
        aux = jnp.float32(NUM_EXPERTS) * jnp.sum(
            (cnt_ref[...] / jnp.float32(BATCH))
            * (psum_ref[...] / jnp.float32(TOKENS)))
        aux_ref[...] = jnp.reshape(aux, (1, 1))


def kernel(x, W):
    xt = x.reshape(TOKENS, HIDDEN)
    grid = TOKENS // TILE_T
    vals, idxs, aux = pl.pallas_call(
        _router_body,
        grid=(grid,),
        in_specs=[
            pl.BlockSpec((TILE_T, HIDDEN), lambda i: (i, 0)),
            pl.BlockSpec((NUM_EXPERTS, HIDDEN), lambda i: (0, 0)),
        ],
        out_specs=[
            pl.BlockSpec((TILE_T, TOP_K), lambda i: (i, 0)),
            pl.BlockSpec((TILE_T, TOP_K), lambda i: (i, 0)),
            pl.BlockSpec((1, 1), lambda i: (0, 0)),
        ],
        out_shape=[
            jax.ShapeDtypeStruct((TOKENS, TOP_K), jnp.float32),
            jax.ShapeDtypeStruct((TOKENS, TOP_K), jnp.int32),
            jax.ShapeDtypeStruct((1, 1), jnp.float32),
        ],
        scratch_shapes=[
            pltpu.VMEM((NUM_EXPERTS, 1), jnp.float32),
            pltpu.VMEM((NUM_EXPERTS, 1), jnp.float32),
            pltpu.VMEM((NUM_EXPERTS, TILE_T), jnp.float32),
        ],
        compiler_params=pltpu.CompilerParams(
            dimension_semantics=("arbitrary",),
        ),
    )(xt, W)
    return (vals.reshape(BATCH, SEQ, TOP_K),
            idxs.reshape(BATCH, SEQ, TOP_K),
            aux[0, 0])
